# Initial kernel scaffold; baseline (speedup 1.0000x reference)
import jax
import jax.numpy as jnp
from jax.experimental import pallas as pl


def kernel(z2, x_pos, edge_index, params):
    raise NotImplementedError("write your pallas kernel here")


# speedup vs baseline: 3.8061x; 3.8061x over previous
import functools

import jax
import jax.numpy as jnp
from jax import lax
from jax.experimental import pallas as pl
from jax.experimental.pallas import tpu as pltpu
from jax.experimental.pallas import tpu_sc as plsc

N = 10000
NP = 10240
E = 320000
NW = 32
K = 128
EP = 327680
NSUB = EP // (NW * K)
RT = NP // 16
BROWS = 1280
NB = NP // BROWS
HDIMS = [7, 32, 64, 128, 256, 256, 512, 512, 256, 256, 128, 64, 32, 16, 3]
FOUT = HDIMS[1:13]


FW = 128


def _chunks(fo):
    f = min(fo, 128)
    return fo // f, f


@functools.cache
def _mesh():
    return plsc.VectorSubcoreMesh(core_axis_name="c", subcore_axis_name="s")


def _zero_vmem16(buf, f):
    zero = jnp.zeros((16,), jnp.float32)
    for i in range(16):
        for j in range(f // 16):
            buf[i, pl.ds(j * 16, 16)] = zero


def _sc_deg(col3):
    fdeg = 16

    def body(col_hbm, out_hbm, colv, ones_v, zbuf, zsh):
        c = lax.axis_index("c")
        s = lax.axis_index("s")
        wid = s * 2 + c
        pltpu.sync_copy(col_hbm.at[wid], colv)
        _zero_vmem16(zbuf, fdeg)
        one = jnp.ones((16,), jnp.float32)
        for i in range(K):
            ones_v[i, pl.ds(0, 16)] = one
        base = s * RT

        def zb(k, _):
            pltpu.sync_copy(zbuf, zsh.at[pl.ds(base + k * 16, 16)])
            return 0

        lax.fori_loop(0, RT // 16, zb, 0)
        plsc.subcore_barrier()

        def step(j, _):
            pltpu.sync_copy(ones_v, zsh.at[colv.at[j]], add=True)
            return 0

        lax.fori_loop(0, NSUB, step, 0)
        plsc.subcore_barrier()
        pltpu.sync_copy(zsh.at[pl.ds(base, RT)], out_hbm.at[c, pl.ds(base, RT)])

    k = pl.kernel(
        body,
        out_type=jax.ShapeDtypeStruct((2, NP, fdeg), jnp.float32),
        mesh=_mesh(),
        scratch_types=[
            pltpu.VMEM((NSUB, K), jnp.int32),
            pltpu.VMEM((K, fdeg), jnp.float32),
            pltpu.VMEM((16, fdeg), jnp.float32),
            pltpu.VMEM_SHARED((NP, fdeg), jnp.float32),
        ],
    )
    return k(col3)


def _sc_segsum(ytab, row3, col3):
    grp = 8
    ngrp = NSUB // grp

    def body(y_hbm, row_hbm, col_hbm, out_hbm, rowv, colv, msg_a, msg_b,
             zbuf, zsh, sem_a, sem_b):
        c = lax.axis_index("c")
        s = lax.axis_index("s")
        wid = s * 2 + c
        base = s * RT

        @pl.when(c == 0)
        def _():
            pltpu.sync_copy(y_hbm.at[pl.ds(base, RT)], zsh.at[pl.ds(base, RT)])

        @pl.when(c == 1)
        def _():
            _zero_vmem16(zbuf, FW)

            def zb(k, _):
                pltpu.sync_copy(zbuf, zsh.at[pl.ds(base + k * 16, 16)])
                return 0

            lax.fori_loop(0, RT // 16, zb, 0)

        plsc.subcore_barrier()

        def group(gg, _):
            pltpu.sync_copy(row_hbm.at[wid, pl.ds(gg * grp, grp)], rowv)
            pltpu.sync_copy(col_hbm.at[wid, pl.ds(gg * grp, grp)], colv)
            pltpu.async_copy(y_hbm.at[rowv.at[0]], msg_a, sem_a)
            pltpu.async_copy(y_hbm.at[rowv.at[1]], msg_b, sem_b)

            def step(jj, _):
                j0 = jj * 2
                pltpu.make_async_copy(y_hbm.at[rowv.at[j0]], msg_a,
                                      sem_a).wait()
                pltpu.sync_copy(msg_a, zsh.at[colv.at[j0]], add=True)

                @pl.when(jj + 1 < grp // 2)
                def _():
                    pltpu.async_copy(y_hbm.at[rowv.at[j0 + 2]], msg_a, sem_a)

                pltpu.make_async_copy(y_hbm.at[rowv.at[j0 + 1]], msg_b,
                                      sem_b).wait()
                pltpu.sync_copy(msg_b, zsh.at[colv.at[j0 + 1]], add=True)

                @pl.when(jj + 1 < grp // 2)
                def _():
                    pltpu.async_copy(y_hbm.at[rowv.at[j0 + 3]], msg_b, sem_b)

                return 0

            lax.fori_loop(0, grp // 2, step, 0)
            return 0

        lax.fori_loop(0, ngrp, group, 0)
        plsc.subcore_barrier()
        pltpu.sync_copy(zsh.at[pl.ds(base, RT)], out_hbm.at[c, pl.ds(base, RT)])

    k = pl.kernel(
        body,
        out_type=jax.ShapeDtypeStruct((2, NP, FW), jnp.float32),
        mesh=_mesh(),
        scratch_types=[
            pltpu.VMEM((grp, K), jnp.int32),
            pltpu.VMEM((grp, K), jnp.int32),
            pltpu.VMEM((K, FW), jnp.float32),
            pltpu.VMEM((K, FW), jnp.float32),
            pltpu.VMEM((16, FW), jnp.float32),
            pltpu.VMEM_SHARED((NP, FW), jnp.float32),
            pltpu.SemaphoreType.DMA,
            pltpu.SemaphoreType.DMA,
        ],
    )
    return k(ytab, row3, col3)


def _lrelu(x):
    return jnp.where(x >= 0, x, 0.01 * x)


def _tc_first(z2p, w1p, degz):
    def body(z_ref, w_ref, d_ref, yo_ref, dis_ref):
        deg = d_ref[0][:, 0:1] + d_ref[1][:, 0:1] + 1.0
        dis = lax.rsqrt(deg)
        dis_ref[...] = jnp.broadcast_to(dis, (BROWS, 8))
        xw = jnp.dot(z_ref[...], w_ref[...], preferred_element_type=jnp.float32)
        yo_ref[...] = jnp.concatenate(
            [dis * xw, jnp.zeros((BROWS, FW - 32), jnp.float32)], axis=1)

    return pl.pallas_call(
        body,
        grid=(NB,),
        in_specs=[
            pl.BlockSpec((BROWS, 8), lambda r: (r, 0)),
            pl.BlockSpec((8, 32), lambda r: (0, 0)),
            pl.BlockSpec((2, BROWS, 16), lambda r: (0, r, 0)),
        ],
        out_specs=[
            pl.BlockSpec((BROWS, FW), lambda r: (r, 0)),
            pl.BlockSpec((BROWS, 8), lambda r: (r, 0)),
        ],
        out_shape=[
            jax.ShapeDtypeStruct((NP, FW), jnp.float32),
            jax.ShapeDtypeStruct((NP, 8), jnp.float32),
        ],
    )(z2p, w1p, degz)


def _tc_layer(zcs, dis8, b, g, be, w, nc_out, f_out):
    nc_in = len(zcs)
    f_in = min(w.shape[0] // nc_in, FW)
    fo_prev = nc_in * f_in
    fo_next = nc_out * f_out

    def body(*refs):
        zr = refs[0:nc_in]
        d_ref, b_ref, g_ref, be_ref, w_ref = refs[nc_in:nc_in + 5]
        outs = refs[nc_in + 5:nc_in + 5 + nc_out]
        t_scr, s_scr = refs[-2:]
        p = pl.program_id(0)
        r = pl.program_id(1)
        dis = d_ref[:, 0:1]

        @pl.when(p == 0)
        def _():
            rows = r * BROWS + lax.broadcasted_iota(jnp.int32, (BROWS, 1), 0)
            msk = (rows < N).astype(jnp.float32)
            for kk in range(nc_in):
                sl = slice(kk * f_in, (kk + 1) * f_in)
                t = (dis * (zr[kk][0][:, :f_in] + zr[kk][1][:, :f_in])
                     + b_ref[0:1, sl])
                t_scr[pl.ds(r * BROWS, BROWS), sl] = t
                tm = t * msk
                s1 = jnp.sum(tm, axis=0, keepdims=True)
                s2 = jnp.sum(tm * tm, axis=0, keepdims=True)

                @pl.when(r == 0)
                def _():
                    s_scr[0:1, sl] = s1
                    s_scr[1:2, sl] = s2

                @pl.when(r > 0)
                def _():
                    s_scr[0:1, sl] = s_scr[0:1, sl] + s1
                    s_scr[1:2, sl] = s_scr[1:2, sl] + s2
            for kk in range(nc_out):
                outs[kk][...] = jnp.zeros((BROWS, FW), jnp.float32)

        @pl.when(p == 1)
        def _():
            m = s_scr[0:1, :] * (1.0 / N)
            ex2 = s_scr[1:2, :] * (1.0 / N)
            inv = lax.rsqrt(ex2 - m * m + 1e-5)
            t = t_scr[pl.ds(r * BROWS, BROWS), :]
            xn = _lrelu((t - m) * inv * g_ref[0:1, :] + be_ref[0:1, :])
            xw = jnp.dot(xn, w_ref[...], preferred_element_type=jnp.float32)
            for kk in range(nc_out):
                val = dis * xw[:, kk * f_out:(kk + 1) * f_out]
                if f_out < FW:
                    val = jnp.concatenate(
                        [val, jnp.zeros((BROWS, FW - f_out), jnp.float32)],
                        axis=1)
                outs[kk][...] = val

    in_specs = (
        [pl.BlockSpec((2, BROWS, FW), lambda p, r: (0, r, 0))] * nc_in
        + [
            pl.BlockSpec((BROWS, 8), lambda p, r: (r, 0)),
            pl.BlockSpec((1, fo_prev), lambda p, r: (0, 0)),
            pl.BlockSpec((1, fo_prev), lambda p, r: (0, 0)),
            pl.BlockSpec((1, fo_prev), lambda p, r: (0, 0)),
            pl.BlockSpec((fo_prev, fo_next), lambda p, r: (0, 0)),
        ]
    )
    out = pl.pallas_call(
        body,
        grid=(2, NB),
        in_specs=in_specs,
        out_specs=[pl.BlockSpec((BROWS, FW), lambda p, r: (r, 0))] * nc_out,
        out_shape=[jax.ShapeDtypeStruct((NP, FW), jnp.float32)] * nc_out,
        scratch_shapes=[
            pltpu.VMEM((NP, fo_prev), jnp.float32),
            pltpu.VMEM((8, fo_prev), jnp.float32),
        ],
    )(*zcs, dis8, b, g, be, w)
    return out


def _tc_final(zc, dis8, b, g, be, w1, b1, w2p, b2p):
    def body(z_ref, d_ref, b_ref, g_ref, be_ref, w1_ref, b1_ref, w2_ref,
             b2_ref, out_ref, t_scr, s_scr):
        p = pl.program_id(0)
        r = pl.program_id(1)
        dis = d_ref[:, 0:1]

        @pl.when(p == 0)
        def _():
            rows = r * BROWS + lax.broadcasted_iota(jnp.int32, (BROWS, 1), 0)
            msk = (rows < N).astype(jnp.float32)
            t = dis * (z_ref[0][:, :32] + z_ref[1][:, :32]) + b_ref[0:1, :]
            t_scr[pl.ds(r * BROWS, BROWS), :] = t
            tm = t * msk
            s1 = jnp.sum(tm, axis=0, keepdims=True)
            s2 = jnp.sum(tm * tm, axis=0, keepdims=True)

            @pl.when(r == 0)
            def _():
                s_scr[0:1, :] = s1
                s_scr[1:2, :] = s2

            @pl.when(r > 0)
            def _():
                s_scr[0:1, :] = s_scr[0:1, :] + s1
                s_scr[1:2, :] = s_scr[1:2, :] + s2
            out_ref[...] = jnp.zeros((BROWS, 128), jnp.float32)

        @pl.when(p == 1)
        def _():
            m = s_scr[0:1, :] * (1.0 / N)
            ex2 = s_scr[1:2, :] * (1.0 / N)
            inv = lax.rsqrt(ex2 - m * m + 1e-5)
            t = t_scr[pl.ds(r * BROWS, BROWS), :]
            xn = _lrelu((t - m) * inv * g_ref[0:1, :] + be_ref[0:1, :])
            h = _lrelu(jnp.dot(xn, w1_ref[...],
                               preferred_element_type=jnp.float32)
                       + b1_ref[0:1, :])
            d = jnp.tanh(jnp.dot(h, w2_ref[...],
                                 preferred_element_type=jnp.float32)
                         + b2_ref[0:1, :])
            nrm = jnp.sqrt(jnp.sum(d * d, axis=1, keepdims=True))
            out_ref[...] = d * (1.0 / (nrm + 1e-12))

    return pl.pallas_call(
        body,
        grid=(2, NB),
        in_specs=[
            pl.BlockSpec((2, BROWS, FW), lambda p, r: (0, r, 0)),
            pl.BlockSpec((BROWS, 8), lambda p, r: (r, 0)),
            pl.BlockSpec((1, 32), lambda p, r: (0, 0)),
            pl.BlockSpec((1, 32), lambda p, r: (0, 0)),
            pl.BlockSpec((1, 32), lambda p, r: (0, 0)),
            pl.BlockSpec((32, 16), lambda p, r: (0, 0)),
            pl.BlockSpec((1, 16), lambda p, r: (0, 0)),
            pl.BlockSpec((16, 128), lambda p, r: (0, 0)),
            pl.BlockSpec((1, 128), lambda p, r: (0, 0)),
        ],
        out_specs=pl.BlockSpec((BROWS, 128), lambda p, r: (r, 0)),
        out_shape=jax.ShapeDtypeStruct((NP, 128), jnp.float32),
        scratch_shapes=[
            pltpu.VMEM((NP, 32), jnp.float32),
            pltpu.VMEM((8, 32), jnp.float32),
        ],
    )(zc, dis8, b, g, be, w1, b1, w2p, b2p)


def kernel(z2, x_pos, edge_index, params):
    del x_pos
    i32 = jnp.int32
    row0 = edge_index[0].astype(i32)
    col0 = edge_index[1].astype(i32)
    rowp = jnp.concatenate([row0, jnp.zeros((EP - E,), i32)])
    colp = jnp.concatenate([col0, jnp.full((EP - E,), N, i32)])
    row3 = rowp.reshape(NW, NSUB, K)
    col3 = colp.reshape(NW, NSUB, K)

    z2p = jnp.pad(z2, ((0, NP - N), (0, 1)))
    w1p = jnp.pad(params["W1"], ((0, 1), (0, 0)))

    degz = _sc_deg(col3)
    y, dis8 = _tc_first(z2p, w1p, degz)
    ycs = [y]

    for i in range(1, 13):
        nc_in, f_in = _chunks(FOUT[i - 1])
        zcs = [_sc_segsum(ycs[kk], row3, col3) for kk in range(nc_in)]
        b = params[f"b{i}"].reshape(1, -1)
        g = params[f"g{i}"].reshape(1, -1)
        be = params[f"be{i}"].reshape(1, -1)
        if i < 12:
            nc_out, f_out = _chunks(FOUT[i])
            ycs = _tc_layer(zcs, dis8, b, g, be, params[f"W{i+1}"],
                            nc_out, f_out)
        else:
            w2p = jnp.pad(params["lin2_W"], ((0, 0), (0, 125)))
            b2p = jnp.pad(params["lin2_b"], (0, 125)).reshape(1, -1)
            out = _tc_final(zcs[0], dis8, b, g, be,
                            params["lin1_W"], params["lin1_b"].reshape(1, -1),
                            w2p, b2p)
    return out[:N, :3]
